# Initial kernel scaffold; baseline (speedup 1.0000x reference)
import functools

import jax
import jax.numpy as jnp
from jax import lax
from jax.experimental import pallas as pl
from jax.experimental.pallas import tpu as pltpu
from jax.experimental.pallas import tpu_sc as plsc

D = 64
MAX_LEN = 5000
B, T = 4096, 200
ROWS = B * T
NC, NS = 2, 16
NW = NC * NS
RPW = ROWS // NW
CHUNK = 512
NCHUNK = RPW // CHUNK
IDXB = 128
NGATHER = CHUNK // IDXB

_mesh = plsc.VectorSubcoreMesh(core_axis_name="c", subcore_axis_name="s")


@functools.partial(
    pl.kernel,
    out_type=jax.ShapeDtypeStruct((ROWS, D), jnp.float32),
    mesh=_mesh,
    scratch_types=[
        pltpu.VMEM((CHUNK,), jnp.float32),
        pltpu.VMEM((NGATHER, IDXB), jnp.int32),
        pltpu.VMEM((CHUNK, D), jnp.float32),
        pltpu.VMEM((CHUNK, D), jnp.float32),
        pltpu.SemaphoreType.DMA,
        pltpu.SemaphoreType.DMA,
    ],
)
def _sc_add_pe(x_hbm, tf_hbm, pe_hbm, out_hbm, tf_v, idx_v, x_v, pe_v,
               sem_x, sem_g):
    wid = lax.axis_index("s") * NC + lax.axis_index("c")
    base = wid * RPW

    def chunk_body(c, carry):
        row0 = base + c * CHUNK
        cp_x = pltpu.async_copy(x_hbm.at[pl.ds(row0, CHUNK)], x_v, sem_x)
        pltpu.sync_copy(tf_hbm.at[pl.ds(row0, CHUNK)], tf_v)

        def idx_body(i, _):
            t = tf_v[pl.ds(i * 16, 16)]
            iv = (t * float(MAX_LEN)).astype(jnp.int32)
            idx_v[i // (IDXB // 16), pl.ds((i % (IDXB // 16)) * 16, 16)] = iv
            return 0

        lax.fori_loop(0, CHUNK // 16, idx_body, 0)

        gathers = []
        for j in range(NGATHER):
            gathers.append(pltpu.async_copy(
                pe_hbm.at[idx_v.at[j]], pe_v.at[pl.ds(j * IDXB, IDXB)], sem_g))
        cp_x.wait()
        for g in gathers:
            g.wait()

        def add_body(r, _):
            for k in range(D // 16):
                s = pl.ds(k * 16, 16)
                x_v[r, s] = x_v[r, s] + pe_v[r, s]
            return 0

        lax.fori_loop(0, CHUNK, add_body, 0)
        pltpu.sync_copy(x_v, out_hbm.at[pl.ds(row0, CHUNK)])
        return carry

    lax.fori_loop(0, NCHUNK, chunk_body, 0)


def kernel(x, time_features, pe):
    out = _sc_add_pe(x.reshape(ROWS, D), time_features.reshape(ROWS), pe)
    return out.reshape(B, T, D)


# speedup vs baseline: 2.5767x; 2.5767x over previous
import functools

import jax
import jax.numpy as jnp
from jax import lax
from jax.experimental import pallas as pl
from jax.experimental.pallas import tpu as pltpu
from jax.experimental.pallas import tpu_sc as plsc

D = 64
MAX_LEN = 5000
B, T = 4096, 200
ROWS = B * T
NC, NS = 2, 16
NW = NC * NS
RPW = ROWS // NW
CHUNK = 512
NCHUNK = RPW // CHUNK
IDXB = 128
NGATHER = CHUNK // IDXB

_mesh = plsc.VectorSubcoreMesh(core_axis_name="c", subcore_axis_name="s")


@functools.partial(
    pl.kernel,
    out_type=jax.ShapeDtypeStruct((ROWS, D), jnp.float32),
    mesh=_mesh,
    scratch_types=[
        pltpu.VMEM((CHUNK,), jnp.float32),
        pltpu.VMEM((NGATHER, IDXB), jnp.int32),
        pltpu.VMEM((CHUNK, D), jnp.float32),
        pltpu.VMEM((CHUNK, D), jnp.float32),
        pltpu.SemaphoreType.DMA,
        pltpu.SemaphoreType.DMA,
    ],
    compiler_params=pltpu.CompilerParams(use_tc_tiling_on_sc=False),
)
def _sc_add_pe(x_hbm, tf_hbm, pe_hbm, out_hbm, tf_v, idx_v, x_v, pe_v,
               sem_x, sem_g):
    wid = lax.axis_index("s") * NC + lax.axis_index("c")
    base = wid * RPW

    def chunk_body(c, carry):
        row0 = base + c * CHUNK
        cp_x = pltpu.async_copy(x_hbm.at[pl.ds(row0, CHUNK)], x_v, sem_x)
        pltpu.sync_copy(tf_hbm.at[pl.ds(row0, CHUNK)], tf_v)

        def idx_body(i, _):
            t = tf_v[pl.ds(i * 16, 16)]
            iv = (t * float(MAX_LEN)).astype(jnp.int32)
            idx_v[i // (IDXB // 16), pl.ds((i % (IDXB // 16)) * 16, 16)] = iv
            return 0

        lax.fori_loop(0, CHUNK // 16, idx_body, 0)

        gathers = []
        for j in range(NGATHER):
            gathers.append(pltpu.async_copy(
                pe_hbm.at[idx_v.at[j]], pe_v.at[pl.ds(j * IDXB, IDXB)], sem_g))
        cp_x.wait()
        for g in gathers:
            g.wait()

        def add_body(r, _):
            for k in range(D // 16):
                s = pl.ds(k * 16, 16)
                x_v[r, s] = x_v[r, s] + pe_v[r, s]
            return 0

        lax.fori_loop(0, CHUNK, add_body, 0)
        pltpu.sync_copy(x_v, out_hbm.at[pl.ds(row0, CHUNK)])
        return carry

    lax.fori_loop(0, NCHUNK, chunk_body, 0)


def kernel(x, time_features, pe):
    out = _sc_add_pe(x.reshape(ROWS, D), time_features.reshape(ROWS), pe)
    return out.reshape(B, T, D)


# speedup vs baseline: 2.5882x; 1.0045x over previous
import functools

import jax
import jax.numpy as jnp
from jax import lax
from jax.experimental import pallas as pl
from jax.experimental.pallas import tpu as pltpu
from jax.experimental.pallas import tpu_sc as plsc

D = 64
MAX_LEN = 5000
B, T = 4096, 200
ROWS = B * T
NC, NS = 2, 16
NW = NC * NS
RPW = ROWS // NW
CHUNK = 512
NCHUNK = RPW // CHUNK
IDXB = 128
NGATHER = CHUNK // IDXB

_mesh = plsc.VectorSubcoreMesh(core_axis_name="c", subcore_axis_name="s")


@functools.partial(
    pl.kernel,
    out_type=jax.ShapeDtypeStruct((ROWS * D,), jnp.float32),
    mesh=_mesh,
    scratch_types=[
        pltpu.VMEM((CHUNK,), jnp.float32),
        pltpu.VMEM((NGATHER, IDXB), jnp.int32),
        pltpu.VMEM((CHUNK * D,), jnp.float32),
        pltpu.VMEM((CHUNK, D), jnp.float32),
        pltpu.SemaphoreType.DMA,
        pltpu.SemaphoreType.DMA,
    ],
    compiler_params=pltpu.CompilerParams(use_tc_tiling_on_sc=False),
)
def _sc_add_pe(x_hbm, tf_hbm, pe_hbm, out_hbm, tf_v, idx_v, x_v, pe_v,
               sem_x, sem_g):
    wid = lax.axis_index("s") * NC + lax.axis_index("c")
    base = wid * RPW

    def chunk_body(c, carry):
        row0 = base + c * CHUNK
        cp_x = pltpu.async_copy(
            x_hbm.at[pl.ds(row0 * D, CHUNK * D)], x_v, sem_x)
        pltpu.sync_copy(tf_hbm.at[pl.ds(row0, CHUNK)], tf_v)

        def idx_body(i, _):
            t = tf_v[pl.ds(i * 16, 16)]
            iv = (t * float(MAX_LEN)).astype(jnp.int32)
            idx_v[i // (IDXB // 16), pl.ds((i % (IDXB // 16)) * 16, 16)] = iv
            return 0

        lax.fori_loop(0, CHUNK // 16, idx_body, 0)

        gathers = []
        for j in range(NGATHER):
            gathers.append(pltpu.async_copy(
                pe_hbm.at[idx_v.at[j]], pe_v.at[pl.ds(j * IDXB, IDXB)], sem_g))
        cp_x.wait()
        for g in gathers:
            g.wait()

        def add_body(r, _):
            for k in range(D // 16):
                x_v[pl.ds(r * D + k * 16, 16)] = (
                    x_v[pl.ds(r * D + k * 16, 16)] + pe_v[r, pl.ds(k * 16, 16)])
            return 0

        lax.fori_loop(0, CHUNK, add_body, 0)
        pltpu.sync_copy(x_v, out_hbm.at[pl.ds(row0 * D, CHUNK * D)])
        return carry

    lax.fori_loop(0, NCHUNK, chunk_body, 0)


def kernel(x, time_features, pe):
    out = _sc_add_pe(x.reshape(ROWS * D), time_features.reshape(ROWS), pe)
    return out.reshape(B, T, D)
